# Initial kernel scaffold; baseline (speedup 1.0000x reference)
import jax
import jax.numpy as jnp
from jax.experimental import pallas as pl


def kernel(x, table, W, b, gamma, beta, moving_mean, moving_var):
    raise NotImplementedError("write your pallas kernel here")


# speedup vs baseline: 2.6869x; 2.6869x over previous
import functools

import jax
import jax.numpy as jnp
from jax import lax
from jax.experimental import pallas as pl
from jax.experimental.pallas import tpu as pltpu
from jax.experimental.pallas import tpu_sc as plsc

DIM = 32
B = 16384
L = 50

NC = 2
NS = 16
NW = NC * NS
E_W = B // NW
CHUNK_E = 32
N_CHUNK = E_W // CHUNK_E
ROWS_C = CHUNK_E * L
DMA_ROWS = 80
N_DMA = ROWS_C // DMA_ROWS


def _pool_body(x_hbm, table_hbm, out_hbm, idx_v, rows_v, pooled_v, gsem):
    wid = lax.axis_index("s") * NC + lax.axis_index("c")
    ebase = wid * E_W
    for c in range(N_CHUNK):
        e0 = ebase + c * CHUNK_E
        pltpu.sync_copy(x_hbm.at[pl.ds(e0 * L, ROWS_C)], idx_v)
        handles = []
        for j in range(N_DMA):
            handles.append(pltpu.async_copy(
                table_hbm.at[idx_v.at[pl.ds(j * DMA_ROWS, DMA_ROWS)]],
                rows_v.at[pl.ds(j * DMA_ROWS, DMA_ROWS)],
                gsem))
        for h in handles:
            h.wait()

        def elem_body(e, carry):
            base = e * L
            acc0 = rows_v[base, pl.ds(0, 16)]
            acc1 = rows_v[base, pl.ds(16, 16)]
            for r in range(1, L):
                acc0 = acc0 + rows_v[base + r, pl.ds(0, 16)]
                acc1 = acc1 + rows_v[base + r, pl.ds(16, 16)]
            pooled_v[e, pl.ds(0, 16)] = acc0 * (1.0 / L)
            pooled_v[e, pl.ds(16, 16)] = acc1 * (1.0 / L)
            return carry

        lax.fori_loop(0, CHUNK_E, elem_body, 0)
        pltpu.sync_copy(pooled_v, out_hbm.at[pl.ds(e0, CHUNK_E)])


_pool = functools.partial(
    pl.kernel,
    mesh=plsc.VectorSubcoreMesh(core_axis_name="c", subcore_axis_name="s"),
    out_type=jax.ShapeDtypeStruct((B, DIM), jnp.float32),
    scratch_types=[
        pltpu.VMEM((ROWS_C,), jnp.int32),
        pltpu.VMEM((ROWS_C, DIM), jnp.float32),
        pltpu.VMEM((CHUNK_E, DIM), jnp.float32),
        pltpu.SemaphoreType.DMA,
    ],
    compiler_params=pltpu.CompilerParams(use_tc_tiling_on_sc=False),
)(_pool_body)


def _tail_body(pooled_ref, w_ref, b_ref, gamma_ref, beta_ref, mean_ref,
               var_ref, out_ref):
    p = pooled_ref[...]
    h = jnp.dot(p, w_ref[...], preferred_element_type=jnp.float32) + b_ref[...]
    scale = gamma_ref[...] * lax.rsqrt(var_ref[...] + 1e-3)
    h = (h - mean_ref[...]) * scale + beta_ref[...]
    nrm = lax.rsqrt(jnp.maximum(jnp.sum(h * h, axis=1, keepdims=True), 1e-12))
    out_ref[...] = h * nrm


def _tail(pooled, w, b, gamma, beta, mean, var):
    blk = 2048
    vec = pl.BlockSpec((1, DIM), lambda i: (0, 0))
    return pl.pallas_call(
        _tail_body,
        grid=(B // blk,),
        in_specs=[
            pl.BlockSpec((blk, DIM), lambda i: (i, 0)),
            pl.BlockSpec((DIM, DIM), lambda i: (0, 0)),
            vec, vec, vec, vec, vec,
        ],
        out_specs=pl.BlockSpec((blk, DIM), lambda i: (i, 0)),
        out_shape=jax.ShapeDtypeStruct((B, DIM), jnp.float32),
    )(pooled, w, b, gamma, beta, mean, var)


def kernel(x, table, W, b, gamma, beta, moving_mean, moving_var):
    x_flat = x.reshape(-1).astype(jnp.int32)
    pooled = _pool(x_flat, table)
    r = lambda v: v.reshape(1, DIM)
    return _tail(pooled, W, r(b), r(gamma), r(beta), r(moving_mean),
                 r(moving_var))


# speedup vs baseline: 2.6985x; 1.0043x over previous
import functools

import jax
import jax.numpy as jnp
from jax import lax
from jax.experimental import pallas as pl
from jax.experimental.pallas import tpu as pltpu
from jax.experimental.pallas import tpu_sc as plsc

DIM = 32
B = 16384
L = 50

NC = 2
NS = 16
NW = NC * NS
E_W = B // NW
CHUNK_E = 32
N_CHUNK = E_W // CHUNK_E
ROWS_C = CHUNK_E * L


def _pool_body(x_hbm, table_hbm, out_hbm, idx_v, rows_v, pooled_v, gsem):
    wid = lax.axis_index("s") * NC + lax.axis_index("c")
    ebase = wid * E_W

    def chunk_body(c, carry):
        e0 = ebase + c * CHUNK_E
        pltpu.sync_copy(x_hbm.at[pl.ds(e0, CHUNK_E), :], idx_v)
        handles = []
        for j in range(CHUNK_E):
            handles.append(pltpu.async_copy(
                table_hbm.at[idx_v.at[j]],
                rows_v.at[pl.ds(j * L, L)],
                gsem))
        for h in handles:
            h.wait()

        def elem_body(e, carry2):
            base = e * L
            acc0 = rows_v[base, pl.ds(0, 16)]
            acc1 = rows_v[base, pl.ds(16, 16)]
            for r in range(1, L):
                acc0 = acc0 + rows_v[base + r, pl.ds(0, 16)]
                acc1 = acc1 + rows_v[base + r, pl.ds(16, 16)]
            pooled_v[e, pl.ds(0, 16)] = acc0 * (1.0 / L)
            pooled_v[e, pl.ds(16, 16)] = acc1 * (1.0 / L)
            return carry2

        lax.fori_loop(0, CHUNK_E, elem_body, 0)
        pltpu.sync_copy(pooled_v, out_hbm.at[pl.ds(e0, CHUNK_E)])
        return carry

    lax.fori_loop(0, N_CHUNK, chunk_body, 0)


_pool = functools.partial(
    pl.kernel,
    mesh=plsc.VectorSubcoreMesh(core_axis_name="c", subcore_axis_name="s"),
    out_type=jax.ShapeDtypeStruct((B, DIM), jnp.float32),
    scratch_types=[
        pltpu.VMEM((CHUNK_E, L), jnp.int32),
        pltpu.VMEM((ROWS_C, DIM), jnp.float32),
        pltpu.VMEM((CHUNK_E, DIM), jnp.float32),
        pltpu.SemaphoreType.DMA,
    ],
    compiler_params=pltpu.CompilerParams(use_tc_tiling_on_sc=False),
)(_pool_body)


def _tail_body(pooled_ref, w_ref, b_ref, gamma_ref, beta_ref, mean_ref,
               var_ref, out_ref):
    p = pooled_ref[...]
    h = jnp.dot(p, w_ref[...], preferred_element_type=jnp.float32) + b_ref[...]
    scale = gamma_ref[...] * lax.rsqrt(var_ref[...] + 1e-3)
    h = (h - mean_ref[...]) * scale + beta_ref[...]
    nrm = lax.rsqrt(jnp.maximum(jnp.sum(h * h, axis=1, keepdims=True), 1e-12))
    out_ref[...] = h * nrm


def _tail(pooled, w, b, gamma, beta, mean, var):
    blk = 2048
    vec = pl.BlockSpec((1, DIM), lambda i: (0, 0))
    return pl.pallas_call(
        _tail_body,
        grid=(B // blk,),
        in_specs=[
            pl.BlockSpec((blk, DIM), lambda i: (i, 0)),
            pl.BlockSpec((DIM, DIM), lambda i: (0, 0)),
            vec, vec, vec, vec, vec,
        ],
        out_specs=pl.BlockSpec((blk, DIM), lambda i: (i, 0)),
        out_shape=jax.ShapeDtypeStruct((B, DIM), jnp.float32),
    )(pooled, w, b, gamma, beta, mean, var)


def kernel(x, table, W, b, gamma, beta, moving_mean, moving_var):
    pooled = _pool(x.astype(jnp.int32), table)
    r = lambda v: v.reshape(1, DIM)
    return _tail(pooled, W, r(b), r(gamma), r(beta), r(moving_mean),
                 r(moving_var))
